# Initial kernel scaffold; baseline (speedup 1.0000x reference)
import functools

import jax
import jax.numpy as jnp
from jax import lax
from jax.experimental import pallas as pl
from jax.experimental.pallas import tpu as pltpu
from jax.experimental.pallas import tpu_sc as plsc

N = 10000
E = 320000
D_IN = 128
HID = 128
NCLS = 16

NC = 2
NS = 16
NT = NC * NS
CW = 128
NCH = 80
EPT = CW * NCH
E_PAD = NT * EPT
N_PAD = N + 16
ROWS_PER_TILE = N_PAD // NS
OUT_ROWS = N // NS

_MESH = plsc.VectorSubcoreMesh(core_axis_name="c", subcore_axis_name="s")


def _deg_kernel(dst_r, ones, zeros, out, dst_i, buf, cnt, sem):
    c = lax.axis_index("c")
    s = lax.axis_index("s")
    wid = s * NC + c
    pltpu.sync_copy(dst_r.at[wid], dst_i)
    pltpu.sync_copy(ones, buf)
    pltpu.sync_copy(zeros, cnt.at[pl.ds(s * ROWS_PER_TILE, ROWS_PER_TILE)])
    plsc.subcore_barrier()

    def step(g, carry):
        pltpu.sync_copy(buf, cnt.at[dst_i.at[g]], add=True)
        return carry

    lax.fori_loop(0, NCH, step, 0)
    plsc.subcore_barrier()
    pltpu.sync_copy(cnt.at[pl.ds(s * OUT_ROWS, OUT_ROWS)],
                    out.at[c, pl.ds(s * OUT_ROWS, OUT_ROWS)])


def _make_deg_call():
    return functools.partial(
        pl.kernel,
        out_type=jax.ShapeDtypeStruct((NC, N, 16), jnp.float32),
        mesh=_MESH,
        scratch_types=[
            pltpu.VMEM((NCH, CW), jnp.int32),
            pltpu.VMEM((CW, 16), jnp.float32),
            pltpu.VMEM_SHARED((N_PAD, 16), jnp.float32),
            pltpu.SemaphoreType.DMA,
        ],
    )(_deg_kernel)


def _agg_kernel(d, table, src_r, dst_r, zeros, out,
                src_i, dst_i, buf0, buf1, acc, sem0, sem1):
    c = lax.axis_index("c")
    s = lax.axis_index("s")
    wid = s * NC + c
    pltpu.sync_copy(src_r.at[wid], src_i)
    pltpu.sync_copy(dst_r.at[wid], dst_i)
    pltpu.sync_copy(zeros, acc.at[pl.ds(s * ROWS_PER_TILE, ROWS_PER_TILE)])
    plsc.subcore_barrier()

    pltpu.async_copy(table.at[src_i.at[0]], buf0, sem0)

    def step(j, carry):
        c0 = 2 * j
        c1 = c0 + 1
        c2 = lax.min(c0 + 2, NCH - 1)
        pltpu.async_copy(table.at[src_i.at[c1]], buf1, sem1)
        pltpu.make_async_copy(table.at[src_i.at[c0]], buf0, sem0).wait()
        pltpu.sync_copy(buf0, acc.at[dst_i.at[c0]], add=True)
        pltpu.async_copy(table.at[src_i.at[c2]], buf0, sem0)
        pltpu.make_async_copy(table.at[src_i.at[c1]], buf1, sem1).wait()
        pltpu.sync_copy(buf1, acc.at[dst_i.at[c1]], add=True)
        return carry

    lax.fori_loop(0, NCH // 2, step, 0)
    pltpu.make_async_copy(table.at[src_i.at[0]], buf0, sem0).wait()
    plsc.subcore_barrier()
    pltpu.sync_copy(acc.at[pl.ds(s * OUT_ROWS, OUT_ROWS)],
                    out.at[c, pl.ds(s * OUT_ROWS, OUT_ROWS)])


def _make_agg_call(d):
    return functools.partial(
        pl.kernel,
        out_type=jax.ShapeDtypeStruct((NC, N, d), jnp.float32),
        mesh=_MESH,
        scratch_types=[
            pltpu.VMEM((NCH, CW), jnp.int32),
            pltpu.VMEM((NCH, CW), jnp.int32),
            pltpu.VMEM((CW, d), jnp.float32),
            pltpu.VMEM((CW, d), jnp.float32),
            pltpu.VMEM_SHARED((N_PAD, d), jnp.float32),
            pltpu.SemaphoreType.DMA,
            pltpu.SemaphoreType.DMA,
        ],
    )(functools.partial(_agg_kernel, d))


BLK = 1000


def _t1_kernel(x_ref, w1_ref, degp_ref, hs1_ref, dinv_ref):
    dp = degp_ref[...]
    dinv = lax.rsqrt(dp[0] + dp[1] + 1.0)
    h = jnp.dot(x_ref[...], w1_ref[...], preferred_element_type=jnp.float32)
    hs1_ref[...] = h * dinv[:, 0:1]
    dinv_ref[...] = dinv


def _t2_kernel(p_ref, hs1_ref, dinv_ref, b1_ref, w2_ref, hs2_ref):
    p = p_ref[...]
    hs1 = hs1_ref[...]
    dinv = dinv_ref[...][:, 0:1]
    out1 = (p[0] + p[1] + hs1) * dinv + b1_ref[...]
    h = jnp.maximum(out1, 0.0)
    h2 = jnp.dot(h, w2_ref[...], preferred_element_type=jnp.float32)
    hs2_ref[...] = h2 * dinv


def _t3_kernel(q_ref, hs2_ref, dinv_ref, b2_ref, out_ref):
    q = q_ref[...]
    dinv = dinv_ref[...][:, 0:1]
    o = (q[0] + q[1] + hs2_ref[...]) * dinv + b2_ref[...]
    m = jnp.max(o, axis=1, keepdims=True)
    e = jnp.exp(o - m)
    lse = jnp.log(jnp.sum(e, axis=1, keepdims=True)) + m
    out_ref[...] = o - lse


def _row_spec(d):
    return pl.BlockSpec((BLK, d), lambda i: (i, 0))


def _pair_spec(d):
    return pl.BlockSpec((2, BLK, d), lambda i: (0, i, 0))


def _full_spec(shape):
    return pl.BlockSpec(shape, lambda i: (0,) * len(shape))


def _t1_call(x, w1, degp):
    return pl.pallas_call(
        _t1_kernel,
        grid=(N // BLK,),
        in_specs=[_row_spec(D_IN), _full_spec((D_IN, HID)), _pair_spec(16)],
        out_specs=[_row_spec(HID), _row_spec(16)],
        out_shape=[jax.ShapeDtypeStruct((N, HID), jnp.float32),
                   jax.ShapeDtypeStruct((N, 16), jnp.float32)],
    )(x, w1, degp)


def _t2_call(p, hs1, dinv, b1, w2):
    return pl.pallas_call(
        _t2_kernel,
        grid=(N // BLK,),
        in_specs=[_pair_spec(HID), _row_spec(HID), _row_spec(16),
                  _full_spec((1, HID)), _full_spec((HID, NCLS))],
        out_specs=_row_spec(NCLS),
        out_shape=jax.ShapeDtypeStruct((N, NCLS), jnp.float32),
    )(p, hs1, dinv, b1, w2)


def _t3_call(q, hs2, dinv, b2):
    return pl.pallas_call(
        _t3_kernel,
        grid=(N // BLK,),
        in_specs=[_pair_spec(NCLS), _row_spec(NCLS), _row_spec(16),
                  _full_spec((1, NCLS))],
        out_specs=_row_spec(NCLS),
        out_shape=jax.ShapeDtypeStruct((N, NCLS), jnp.float32),
    )(q, hs2, dinv, b2)


def kernel(x, edge_index, W1, b1, W2, b2):
    src = edge_index[0]
    dst = edge_index[1]
    npad = E_PAD - E
    src_r = jnp.concatenate(
        [src, jnp.zeros((npad,), jnp.int32)]).reshape(NT, NCH, CW)
    dst_r = jnp.concatenate(
        [dst, jnp.full((npad,), N, jnp.int32)]).reshape(NT, NCH, CW)

    ones16 = jnp.ones((CW, 16), jnp.float32)
    zeros16 = jnp.zeros((ROWS_PER_TILE, 16), jnp.float32)
    zeros128 = jnp.zeros((ROWS_PER_TILE, HID), jnp.float32)

    degp = _make_deg_call()(dst_r, ones16, zeros16)
    hs1, dinv = _t1_call(x, W1, degp)
    p1 = _make_agg_call(HID)(hs1, src_r, dst_r, zeros128)
    hs2 = _t2_call(p1, hs1, dinv, b1.reshape(1, HID), W2)
    p2 = _make_agg_call(NCLS)(hs2, src_r, dst_r, zeros16)
    return _t3_call(p2, hs2, dinv, b2.reshape(1, NCLS))


# speedup vs baseline: 8.7658x; 8.7658x over previous
import functools

import jax
import jax.numpy as jnp
from jax import lax
from jax.experimental import pallas as pl
from jax.experimental.pallas import tpu as pltpu
from jax.experimental.pallas import tpu_sc as plsc

N = 10000
E = 320000
D_IN = 128
HID = 128
NCLS = 16

NC = 2
NS = 16
NT = NC * NS
CW = 128
NCH = 80
NCH_H = NCH // 2
EPT = CW * NCH
E_PAD = NT * EPT
NP = 10240
ROWS_PER_TILE = NP // NS

_MESH = plsc.VectorSubcoreMesh(core_axis_name="c", subcore_axis_name="s")


def _deg_kernel(dst_r, ones, zeros, out, dst_i, buf, cnt, sem):
    c = lax.axis_index("c")
    s = lax.axis_index("s")
    wid = s * NC + c
    pltpu.sync_copy(dst_r.at[wid], dst_i)
    pltpu.sync_copy(ones, buf)
    pltpu.sync_copy(zeros, cnt.at[pl.ds(s * ROWS_PER_TILE, ROWS_PER_TILE)])
    plsc.subcore_barrier()

    def step(g, carry):
        pltpu.sync_copy(buf, cnt.at[dst_i.at[g]], add=True)
        return carry

    lax.fori_loop(0, NCH, step, 0)
    plsc.subcore_barrier()
    pltpu.sync_copy(cnt.at[pl.ds(s * ROWS_PER_TILE, ROWS_PER_TILE)],
                    out.at[c, pl.ds(s * ROWS_PER_TILE, ROWS_PER_TILE)])


def _make_deg_call():
    return functools.partial(
        pl.kernel,
        out_type=jax.ShapeDtypeStruct((NC, NP, HID), jnp.float32),
        mesh=_MESH,
        scratch_types=[
            pltpu.VMEM((NCH, CW), jnp.int32),
            pltpu.VMEM((CW, HID), jnp.float32),
            pltpu.VMEM_SHARED((NP, HID), jnp.float32),
            pltpu.SemaphoreType.DMA,
        ],
    )(_deg_kernel)


def _agg_kernel(d, table, src_r, dst_r, zeros, out,
                src_i, dst_i, buf0, buf1, acc, sem0, sem1):
    c = lax.axis_index("c")
    s = lax.axis_index("s")
    wid = s * NC + c
    pltpu.sync_copy(zeros, acc.at[pl.ds(s * ROWS_PER_TILE, ROWS_PER_TILE)])
    plsc.subcore_barrier()

    def step(j, carry):
        c0 = 2 * j
        c1 = c0 + 1
        c2 = lax.min(c0 + 2, NCH_H - 1)
        pltpu.async_copy(table.at[src_i.at[c1]], buf1, sem1)
        pltpu.make_async_copy(table.at[src_i.at[c0]], buf0, sem0).wait()
        pltpu.sync_copy(buf0, acc.at[dst_i.at[c0]], add=True)
        pltpu.async_copy(table.at[src_i.at[c2]], buf0, sem0)
        pltpu.make_async_copy(table.at[src_i.at[c1]], buf1, sem1).wait()
        pltpu.sync_copy(buf1, acc.at[dst_i.at[c1]], add=True)
        return carry

    for half in range(2):
        pltpu.sync_copy(src_r.at[wid, pl.ds(half * NCH_H, NCH_H)], src_i)
        pltpu.sync_copy(dst_r.at[wid, pl.ds(half * NCH_H, NCH_H)], dst_i)
        pltpu.async_copy(table.at[src_i.at[0]], buf0, sem0)
        lax.fori_loop(0, NCH_H // 2, step, 0)
        pltpu.make_async_copy(table.at[src_i.at[0]], buf0, sem0).wait()
    plsc.subcore_barrier()
    pltpu.sync_copy(acc.at[pl.ds(s * ROWS_PER_TILE, ROWS_PER_TILE)],
                    out.at[c, pl.ds(s * ROWS_PER_TILE, ROWS_PER_TILE)])


def _make_agg_call(d):
    return functools.partial(
        pl.kernel,
        out_type=jax.ShapeDtypeStruct((NC, NP, d), jnp.float32),
        mesh=_MESH,
        scratch_types=[
            pltpu.VMEM((NCH_H, CW), jnp.int32),
            pltpu.VMEM((NCH_H, CW), jnp.int32),
            pltpu.VMEM((CW, d), jnp.float32),
            pltpu.VMEM((CW, d), jnp.float32),
            pltpu.VMEM_SHARED((NP, d), jnp.float32),
            pltpu.SemaphoreType.DMA,
            pltpu.SemaphoreType.DMA,
        ],
    )(functools.partial(_agg_kernel, d))


BLK = 1280


def _t1_kernel(x_ref, w1_ref, degp_ref, hs1_ref, dinv_ref):
    dp = degp_ref[...]
    dinv = lax.rsqrt(dp[0] + dp[1] + 1.0)
    h = jnp.dot(x_ref[...], w1_ref[...], preferred_element_type=jnp.float32)
    hs1_ref[...] = h * dinv[:, 0:1]
    dinv_ref[...] = dinv[:, 0:16]


def _t2_kernel(p_ref, hs1_ref, dinv_ref, b1_ref, w2_ref, hs2_ref):
    p = p_ref[...]
    hs1 = hs1_ref[...]
    dinv = dinv_ref[...][:, 0:1]
    out1 = (p[0] + p[1] + hs1) * dinv + b1_ref[...]
    h = jnp.maximum(out1, 0.0)
    h2 = jnp.dot(h, w2_ref[...], preferred_element_type=jnp.float32)
    hs2_ref[...] = h2 * dinv


def _t3_kernel(q_ref, hs2_ref, dinv_ref, b2_ref, out_ref):
    q = q_ref[...]
    dinv = dinv_ref[...][:, 0:1]
    o_full = (q[0] + q[1] + hs2_ref[...]) * dinv + b2_ref[...]
    o = o_full[:, 0:NCLS]
    m = jnp.max(o, axis=1, keepdims=True)
    e = jnp.exp(o - m)
    lse = jnp.log(jnp.sum(e, axis=1, keepdims=True)) + m
    out_ref[...] = o - lse


def _row_spec(d):
    return pl.BlockSpec((BLK, d), lambda i: (i, 0))


def _pair_spec(d):
    return pl.BlockSpec((2, BLK, d), lambda i: (0, i, 0))


def _full_spec(shape):
    return pl.BlockSpec(shape, lambda i: (0,) * len(shape))


def _t1_call(x, w1, degp):
    return pl.pallas_call(
        _t1_kernel,
        grid=(NP // BLK,),
        in_specs=[_row_spec(D_IN), _full_spec((D_IN, HID)), _pair_spec(HID)],
        out_specs=[_row_spec(HID), _row_spec(16)],
        out_shape=[jax.ShapeDtypeStruct((NP, HID), jnp.float32),
                   jax.ShapeDtypeStruct((NP, 16), jnp.float32)],
    )(x, w1, degp)


def _t2_call(p, hs1, dinv, b1, w2):
    return pl.pallas_call(
        _t2_kernel,
        grid=(NP // BLK,),
        in_specs=[_pair_spec(HID), _row_spec(HID), _row_spec(16),
                  _full_spec((1, HID)), _full_spec((HID, HID))],
        out_specs=_row_spec(HID),
        out_shape=jax.ShapeDtypeStruct((NP, HID), jnp.float32),
    )(p, hs1, dinv, b1, w2)


def _t3_call(q, hs2, dinv, b2):
    return pl.pallas_call(
        _t3_kernel,
        grid=(NP // BLK,),
        in_specs=[_pair_spec(HID), _row_spec(HID), _row_spec(16),
                  _full_spec((1, HID))],
        out_specs=_row_spec(NCLS),
        out_shape=jax.ShapeDtypeStruct((NP, NCLS), jnp.float32),
    )(q, hs2, dinv, b2)


def kernel(x, edge_index, W1, b1, W2, b2):
    src = edge_index[0]
    dst = edge_index[1]
    npad = E_PAD - E
    src_r = jnp.concatenate(
        [src, jnp.zeros((npad,), jnp.int32)]).reshape(NT, NCH, CW)
    dst_r = jnp.concatenate(
        [dst, jnp.full((npad,), N, jnp.int32)]).reshape(NT, NCH, CW)

    ones128 = jnp.ones((CW, HID), jnp.float32)
    zeros128 = jnp.zeros((ROWS_PER_TILE, HID), jnp.float32)
    x_pad = jnp.pad(x, ((0, NP - N), (0, 0)))
    w2_pad = jnp.pad(W2, ((0, 0), (0, HID - NCLS)))
    b2_pad = jnp.pad(b2, (0, HID - NCLS)).reshape(1, HID)

    degp = _make_deg_call()(dst_r, ones128, zeros128)
    hs1, dinv = _t1_call(x_pad, W1, degp)
    agg = _make_agg_call(HID)
    p1 = agg(hs1, src_r, dst_r, zeros128)
    hs2 = _t2_call(p1, hs1, dinv, b1.reshape(1, HID), w2_pad)
    p2 = agg(hs2, src_r, dst_r, zeros128)
    return _t3_call(p2, hs2, dinv, b2_pad)[:N]


# speedup vs baseline: 8.8592x; 1.0107x over previous
import functools

import jax
import jax.numpy as jnp
from jax import lax
from jax.experimental import pallas as pl
from jax.experimental.pallas import tpu as pltpu
from jax.experimental.pallas import tpu_sc as plsc

N = 10000
E = 320000
D_IN = 128
HID = 128
NCLS = 16

NC = 2
NS = 16
NT = NC * NS
CW = 64
NCH = 160
NQ = 4
CPQ = NCH // NQ
NBUF = 4
EPT = CW * NCH
E_PAD = NT * EPT
NP = 10240
ROWS_PER_TILE = NP // NS

_MESH = plsc.VectorSubcoreMesh(core_axis_name="c", subcore_axis_name="s")


def _deg_kernel(dst_r, ones, zeros, out, dst_i, buf, cnt, sem):
    c = lax.axis_index("c")
    s = lax.axis_index("s")
    wid = s * NC + c
    pltpu.sync_copy(dst_r.at[wid], dst_i)
    pltpu.sync_copy(ones, buf)
    pltpu.sync_copy(zeros, cnt.at[pl.ds(s * ROWS_PER_TILE, ROWS_PER_TILE)])
    plsc.subcore_barrier()

    def step(g, carry):
        pltpu.sync_copy(buf, cnt.at[dst_i.at[g]], add=True)
        return carry

    lax.fori_loop(0, NCH, step, 0)
    plsc.subcore_barrier()
    pltpu.sync_copy(cnt.at[pl.ds(s * ROWS_PER_TILE, ROWS_PER_TILE)],
                    out.at[c, pl.ds(s * ROWS_PER_TILE, ROWS_PER_TILE)])


def _make_deg_call():
    return functools.partial(
        pl.kernel,
        out_type=jax.ShapeDtypeStruct((NC, NP, HID), jnp.float32),
        mesh=_MESH,
        scratch_types=[
            pltpu.VMEM((NCH, CW), jnp.int32),
            pltpu.VMEM((CW, HID), jnp.float32),
            pltpu.VMEM_SHARED((NP, HID), jnp.float32),
            pltpu.SemaphoreType.DMA,
        ],
    )(_deg_kernel)


def _agg_kernel(d, table, src_r, dst_r, zeros, out,
                src_i, dst_i, bufs, gsems, ssems, acc):
    c = lax.axis_index("c")
    s = lax.axis_index("s")
    wid = s * NC + c
    pltpu.sync_copy(zeros, acc.at[pl.ds(s * ROWS_PER_TILE, ROWS_PER_TILE)])
    plsc.subcore_barrier()

    for q in range(NQ):
        pltpu.sync_copy(src_r.at[wid, pl.ds(q * CPQ, CPQ)], src_i)
        pltpu.sync_copy(dst_r.at[wid, pl.ds(q * CPQ, CPQ)], dst_i)
        for b in range(NBUF):
            pltpu.async_copy(table.at[src_i.at[b]], bufs[b], gsems[b])

        def step(g, carry):
            for b in range(NBUF):
                cc = NBUF * g + b
                pltpu.make_async_copy(table.at[src_i.at[0]], bufs[b],
                                      gsems[b]).wait()
                pltpu.async_copy(bufs[b], acc.at[dst_i.at[cc]], ssems[b],
                                 add=True)
            for b in range(NBUF):
                cn = NBUF * g + b + NBUF
                pltpu.make_async_copy(bufs[b], acc.at[dst_i.at[0]],
                                      ssems[b]).wait()
                pltpu.async_copy(table.at[src_i.at[cn]], bufs[b], gsems[b])
            return carry

        lax.fori_loop(0, CPQ // NBUF - 1, step, 0)
        for b in range(NBUF):
            cc = CPQ - NBUF + b
            pltpu.make_async_copy(table.at[src_i.at[0]], bufs[b],
                                  gsems[b]).wait()
            pltpu.async_copy(bufs[b], acc.at[dst_i.at[cc]], ssems[b], add=True)
        for b in range(NBUF):
            pltpu.make_async_copy(bufs[b], acc.at[dst_i.at[0]],
                                  ssems[b]).wait()
    plsc.subcore_barrier()
    pltpu.sync_copy(acc.at[pl.ds(s * ROWS_PER_TILE, ROWS_PER_TILE)],
                    out.at[c, pl.ds(s * ROWS_PER_TILE, ROWS_PER_TILE)])


def _make_agg_call(d):
    return functools.partial(
        pl.kernel,
        out_type=jax.ShapeDtypeStruct((NC, NP, d), jnp.float32),
        mesh=_MESH,
        scratch_types=[
            pltpu.VMEM((CPQ, CW), jnp.int32),
            pltpu.VMEM((CPQ, CW), jnp.int32),
            [pltpu.VMEM((CW, d), jnp.float32) for _ in range(NBUF)],
            [pltpu.SemaphoreType.DMA for _ in range(NBUF)],
            [pltpu.SemaphoreType.DMA for _ in range(NBUF)],
            pltpu.VMEM_SHARED((NP, d), jnp.float32),
        ],
    )(functools.partial(_agg_kernel, d))


BLK = 1280


def _t1_kernel(x_ref, w1_ref, degp_ref, hs1_ref, dinv_ref):
    dp = degp_ref[...]
    dinv = lax.rsqrt(dp[0] + dp[1] + 1.0)
    h = jnp.dot(x_ref[...], w1_ref[...], preferred_element_type=jnp.float32)
    hs1_ref[...] = h * dinv[:, 0:1]
    dinv_ref[...] = dinv[:, 0:16]


def _t2_kernel(p_ref, hs1_ref, dinv_ref, b1_ref, w2_ref, hs2_ref):
    p = p_ref[...]
    hs1 = hs1_ref[...]
    dinv = dinv_ref[...][:, 0:1]
    out1 = (p[0] + p[1] + hs1) * dinv + b1_ref[...]
    h = jnp.maximum(out1, 0.0)
    h2 = jnp.dot(h, w2_ref[...], preferred_element_type=jnp.float32)
    hs2_ref[...] = h2 * dinv


def _t3_kernel(q_ref, hs2_ref, dinv_ref, b2_ref, out_ref):
    q = q_ref[...]
    dinv = dinv_ref[...][:, 0:1]
    o_full = (q[0] + q[1] + hs2_ref[...]) * dinv + b2_ref[...]
    o = o_full[:, 0:NCLS]
    m = jnp.max(o, axis=1, keepdims=True)
    e = jnp.exp(o - m)
    lse = jnp.log(jnp.sum(e, axis=1, keepdims=True)) + m
    out_ref[...] = o - lse


def _row_spec(d):
    return pl.BlockSpec((BLK, d), lambda i: (i, 0))


def _pair_spec(d):
    return pl.BlockSpec((2, BLK, d), lambda i: (0, i, 0))


def _full_spec(shape):
    return pl.BlockSpec(shape, lambda i: (0,) * len(shape))


def _t1_call(x, w1, degp):
    return pl.pallas_call(
        _t1_kernel,
        grid=(NP // BLK,),
        in_specs=[_row_spec(D_IN), _full_spec((D_IN, HID)), _pair_spec(HID)],
        out_specs=[_row_spec(HID), _row_spec(16)],
        out_shape=[jax.ShapeDtypeStruct((NP, HID), jnp.float32),
                   jax.ShapeDtypeStruct((NP, 16), jnp.float32)],
    )(x, w1, degp)


def _t2_call(p, hs1, dinv, b1, w2):
    return pl.pallas_call(
        _t2_kernel,
        grid=(NP // BLK,),
        in_specs=[_pair_spec(HID), _row_spec(HID), _row_spec(16),
                  _full_spec((1, HID)), _full_spec((HID, HID))],
        out_specs=_row_spec(HID),
        out_shape=jax.ShapeDtypeStruct((NP, HID), jnp.float32),
    )(p, hs1, dinv, b1, w2)


def _t3_call(q, hs2, dinv, b2):
    return pl.pallas_call(
        _t3_kernel,
        grid=(NP // BLK,),
        in_specs=[_pair_spec(HID), _row_spec(HID), _row_spec(16),
                  _full_spec((1, HID))],
        out_specs=_row_spec(NCLS),
        out_shape=jax.ShapeDtypeStruct((NP, NCLS), jnp.float32),
    )(q, hs2, dinv, b2)


def kernel(x, edge_index, W1, b1, W2, b2):
    src = edge_index[0]
    dst = edge_index[1]
    npad = E_PAD - E
    src_r = jnp.concatenate(
        [src, jnp.zeros((npad,), jnp.int32)]).reshape(NT, NCH, CW)
    dst_r = jnp.concatenate(
        [dst, jnp.full((npad,), N, jnp.int32)]).reshape(NT, NCH, CW)

    ones128 = jnp.ones((CW, HID), jnp.float32)
    zeros128 = jnp.zeros((ROWS_PER_TILE, HID), jnp.float32)
    x_pad = jnp.pad(x, ((0, NP - N), (0, 0)))
    w2_pad = jnp.pad(W2, ((0, 0), (0, HID - NCLS)))
    b2_pad = jnp.pad(b2, (0, HID - NCLS)).reshape(1, HID)

    degp = _make_deg_call()(dst_r, ones128, zeros128)
    hs1, dinv = _t1_call(x_pad, W1, degp)
    agg = _make_agg_call(HID)
    p1 = agg(hs1, src_r, dst_r, zeros128)
    hs2 = _t2_call(p1, hs1, dinv, b1.reshape(1, HID), w2_pad)
    p2 = agg(hs2, src_r, dst_r, zeros128)
    return _t3_call(p2, hs2, dinv, b2_pad)[:N]
